# baseline (speedup 1.0000x reference)
import functools

import jax
import jax.numpy as jnp
from jax import lax
from jax.experimental import pallas as pl
from jax.experimental.pallas import tpu as pltpu
from jax.experimental.pallas import tpu_sc as plsc

EMBED_DIM = 64
NUM_WORKERS = 32
CHUNK = 128


def _cdiv(a, b):
    return (a + b - 1) // b


@functools.partial(jax.jit, static_argnames=("n_rows",))
def _embed_gather(idx2d, table, *, n_rows):
    per_w = n_rows // NUM_WORKERS
    cpw = idx2d.shape[0] // NUM_WORKERS
    full = per_w // CHUNK
    tail = per_w - full * CHUNK
    mesh = plsc.VectorSubcoreMesh(core_axis_name="c", subcore_axis_name="s")

    scratch = [
        pltpu.VMEM((cpw, CHUNK), jnp.int32),
        pltpu.VMEM((CHUNK, EMBED_DIM), jnp.float32),
        pltpu.VMEM((CHUNK, EMBED_DIM), jnp.float32),
        pltpu.SemaphoreType.DMA((2,)),
        pltpu.SemaphoreType.DMA((2,)),
    ]

    @functools.partial(
        pl.kernel,
        mesh=mesh,
        compiler_params=pltpu.CompilerParams(use_tc_tiling_on_sc=False),
        out_type=jax.ShapeDtypeStruct((n_rows, EMBED_DIM), jnp.float32),
        scratch_types=scratch,
    )
    def k(idx_hbm, table_hbm, out_hbm, idx_v, rows_a, rows_b, gsem, osem):
        wid = lax.axis_index("s") * 2 + lax.axis_index("c")
        base = wid * per_w
        bufs = (rows_a, rows_b)
        pltpu.sync_copy(idx_hbm.at[pl.ds(wid * cpw, cpw)], idx_v)

        def gather_start(j, b):
            pltpu.async_copy(
                table_hbm.at[idx_v.at[j]], bufs[b], gsem.at[b]
            )

        def gather_wait(b):
            pltpu.make_async_copy(
                table_hbm.at[pl.ds(0, CHUNK)], bufs[b], gsem.at[b]
            ).wait()

        def out_start(j, b, width):
            pltpu.async_copy(
                bufs[b].at[pl.ds(0, width)],
                out_hbm.at[pl.ds(base + j * CHUNK, width)],
                osem.at[b],
            )

        def out_wait(b, width):
            pltpu.make_async_copy(
                bufs[b].at[pl.ds(0, width)],
                out_hbm.at[pl.ds(base, width)],
                osem.at[b],
            ).wait()

        n_chunks = full + (1 if tail else 0)

        gather_start(0, 0)
        if n_chunks > 1:
            gather_start(1, 1)

        @pl.loop(0, max(n_chunks - 2, 0))
        def _(g):
            b = lax.rem(g, 2)

            def do(b):
                gather_wait(b)
                out_start(g, b, CHUNK)
                out_wait(b, CHUNK)
                gather_start(g + 2, b)

            @pl.when(b == 0)
            def _():
                do(0)

            @pl.when(b == 1)
            def _():
                do(1)

        for t in range(min(2, n_chunks)):
            j = max(n_chunks - 2, 0) + t
            b = j % 2
            width = tail if (tail and j == n_chunks - 1) else CHUNK
            gather_wait(b)
            out_start(j, b, width)
            out_wait(b, width)

    return k(idx2d, table)


def kernel(tokens, emb_weight):
    n = tokens.shape[0]
    per_w = n // NUM_WORKERS
    assert per_w * NUM_WORKERS == n
    cpw = _cdiv(per_w, CHUNK)
    idx = tokens.astype(jnp.int32).reshape(NUM_WORKERS, per_w)
    idx = jnp.pad(idx, ((0, 0), (0, cpw * CHUNK - per_w)))
    idx2d = idx.reshape(NUM_WORKERS * cpw, CHUNK)
    return _embed_gather(idx2d, emb_weight, n_rows=n)
